# baseline (speedup 1.0000x reference)
import functools

import jax
import jax.numpy as jnp
from jax import lax
from jax.experimental import pallas as pl
from jax.experimental.pallas import tpu as pltpu
from jax.experimental.pallas import tpu_sc as plsc

_B = 16384
_EMB = 16
_ROWS = 1000000
_FOLD = 128 // _EMB
_TROWS = _ROWS // _FOLD
_NC = 2
_NS = 16
_NW = _NC * _NS
_BPW = _B // _NW
_CH = 128
_NCH = _BPW // _CH
_NBUF = 4

_mesh = plsc.VectorSubcoreMesh(core_axis_name="c", subcore_axis_name="s")


@functools.partial(
    pl.kernel,
    mesh=_mesh,
    out_type=[
        jax.ShapeDtypeStruct((_B // _FOLD, 128), jnp.float32),
        jax.ShapeDtypeStruct((_B // _FOLD, 128), jnp.float32),
        jax.ShapeDtypeStruct((_B // _FOLD, 128), jnp.float32),
    ],
    scratch_types=[
        pltpu.VMEM((_BPW,), jnp.int32),
        pltpu.VMEM((_BPW,), jnp.int32),
        pltpu.VMEM((_BPW,), jnp.int32),
        pltpu.VMEM((_BPW,), jnp.int32),
        pltpu.VMEM((_BPW,), jnp.int32),
        pltpu.VMEM((_BPW,), jnp.int32),
        [pltpu.VMEM((_CH, 128), jnp.float32)] * _NBUF,
        pltpu.VMEM((_BPW // _FOLD, 128), jnp.float32),
        pltpu.VMEM((_BPW // _FOLD, 128), jnp.float32),
        pltpu.VMEM((_BPW // _FOLD, 128), jnp.float32),
        [pltpu.SemaphoreType.DMA] * _NBUF,
    ],
)
def _sc_gather(uidx_hbm, iidx_hbm, gu_hbm, gi_hbm, mu_hbm, mi_hbm,
               xgmf_hbm, xum_hbm, xim_hbm,
               uidx_v, iidx_v, u8_v, uoff_v, i8_v, ioff_v,
               stages, xg_out, mu_out, mi_out, sems):
    wid = lax.axis_index("s") * _NC + lax.axis_index("c")

    pltpu.sync_copy(uidx_hbm.at[wid], uidx_v)
    pltpu.sync_copy(iidx_hbm.at[wid], iidx_v)

    def prep(m, carry):
        sl = pl.ds(m * 16, 16)
        uv = uidx_v[sl]
        iv = iidx_v[sl]
        u8_v[sl] = uv >> 3
        uoff_v[sl] = (uv & 7) << 4
        i8_v[sl] = iv >> 3
        ioff_v[sl] = (iv & 7) << 4
        return carry

    lax.fori_loop(0, _BPW // 16, prep, 0)

    units = []
    for c in range(_NCH):
        units.append((gu_hbm, u8_v, uoff_v, c, False, xg_out))
        units.append((gi_hbm, i8_v, ioff_v, c, True, xg_out))
        units.append((mu_hbm, u8_v, uoff_v, c, False, mu_out))
        units.append((mi_hbm, i8_v, ioff_v, c, False, mi_out))

    def fire(i):
        table, idx8, _, c, _, _ = units[i]
        return pltpu.async_copy(
            table.at[idx8.at[pl.ds(c * _CH, _CH)]],
            stages[i % _NBUF], sems[i % _NBUF])

    def extract(i):
        _, _, offs, c, mul, out = units[i]
        stage = stages[i % _NBUF]

        def body(g, carry):
            ovec = offs[pl.ds(c * _CH + g * 16, 16)]
            for j in range(16):
                val = stage[g * 16 + j, pl.ds(ovec[j], _EMB)]
                row8 = c * (_CH // _FOLD) + 2 * g + (j // _FOLD)
                col = pl.ds((j % _FOLD) * _EMB, _EMB)
                if mul:
                    out[row8, col] = out[row8, col] * val
                else:
                    out[row8, col] = val
            return carry

        lax.fori_loop(0, _CH // 16, body, 0)

    descs = {i: fire(i) for i in range(_NBUF)}
    for i in range(len(units)):
        descs[i].wait()
        extract(i)
        if i + _NBUF < len(units):
            descs[i + _NBUF] = fire(i + _NBUF)

    base = wid * (_BPW // _FOLD)
    pltpu.sync_copy(xg_out, xgmf_hbm.at[pl.ds(base, _BPW // _FOLD)])
    pltpu.sync_copy(mu_out, xum_hbm.at[pl.ds(base, _BPW // _FOLD)])
    pltpu.sync_copy(mi_out, xim_hbm.at[pl.ds(base, _BPW // _FOLD)])


def _dense_body(xgmf, xum, xim, gen, gW, gb, W1, b1, W2, b2, Wf, bf, out):
    xg = jnp.dot(gen[...], gW[...], preferred_element_type=jnp.float32) + gb[...]
    h = jnp.concatenate([xum[...], xim[...], xg], axis=1)
    h = jnp.maximum(
        jnp.dot(h, W1[...], preferred_element_type=jnp.float32) + b1[...], 0.0)
    h = jnp.maximum(
        jnp.dot(h, W2[...], preferred_element_type=jnp.float32) + b2[...], 0.0)
    wf = Wf[...]
    acc = jnp.dot(xgmf[...], wf[0:_EMB, :], preferred_element_type=jnp.float32)
    acc = acc + jnp.dot(h, wf[_EMB:, :], preferred_element_type=jnp.float32)
    out[...] = acc + bf[...]


_BT = 2048


def _dense(xgmf, xum, xim, gen, gW, gb, W1, b1, W2, b2, Wf, bf):
    grid = (_B // _BT,)
    row = lambda i: (i, 0)
    full = lambda i: (0, 0)
    return pl.pallas_call(
        _dense_body,
        grid=grid,
        in_specs=[
            pl.BlockSpec((_BT, _EMB), row),
            pl.BlockSpec((_BT, _EMB), row),
            pl.BlockSpec((_BT, _EMB), row),
            pl.BlockSpec((_BT, 18), row),
            pl.BlockSpec((18, 16), full),
            pl.BlockSpec((1, 16), full),
            pl.BlockSpec((48, 128), full),
            pl.BlockSpec((1, 128), full),
            pl.BlockSpec((128, 64), full),
            pl.BlockSpec((1, 64), full),
            pl.BlockSpec((80, 1), full),
            pl.BlockSpec((1, 1), full),
        ],
        out_specs=pl.BlockSpec((_BT, 1), row),
        out_shape=jax.ShapeDtypeStruct((_B, 1), jnp.float32),
        compiler_params=pltpu.CompilerParams(
            dimension_semantics=("parallel",)),
    )(xgmf, xum, xim, gen, gW, gb, W1, b1, W2, b2, Wf, bf)


def kernel(user_indices, item_indices, genres_vec, gmf_user_emb, gmf_item_emb,
           mlp_user_emb, mlp_item_emb, genres_W, genres_b, W1, b1, W2, b2,
           Wf, bf):
    u2 = user_indices.astype(jnp.int32).reshape(_NW, _BPW)
    i2 = item_indices.astype(jnp.int32).reshape(_NW, _BPW)
    xg128, mu128, mi128 = _sc_gather(
        u2, i2,
        gmf_user_emb.reshape(_TROWS, 128), gmf_item_emb.reshape(_TROWS, 128),
        mlp_user_emb.reshape(_TROWS, 128), mlp_item_emb.reshape(_TROWS, 128))
    out = _dense(
        xg128.reshape(_B, _EMB), mu128.reshape(_B, _EMB),
        mi128.reshape(_B, _EMB), genres_vec, genres_W,
        genres_b.reshape(1, -1), W1, b1.reshape(1, -1), W2,
        b2.reshape(1, -1), Wf, bf.reshape(1, -1))
    return out[:, 0]


# speedup vs baseline: 1.4136x; 1.4136x over previous
import functools

import jax
import jax.numpy as jnp
from jax import lax
from jax.experimental import pallas as pl
from jax.experimental.pallas import tpu as pltpu
from jax.experimental.pallas import tpu_sc as plsc

_B = 16384
_EMB = 16
_NC = 2
_NS = 16
_NW = _NC * _NS
_BPW = _B // _NW
_UROWS = 128

_mesh = plsc.VectorSubcoreMesh(core_axis_name="c", subcore_axis_name="s")


@functools.partial(
    pl.kernel,
    mesh=_mesh,
    out_type=[
        jax.ShapeDtypeStruct((_B, _EMB), jnp.float32),
        jax.ShapeDtypeStruct((_B, _EMB), jnp.float32),
        jax.ShapeDtypeStruct((_B, _EMB), jnp.float32),
        jax.ShapeDtypeStruct((_B, _EMB), jnp.float32),
    ],
    scratch_types=[
        pltpu.VMEM((_BPW,), jnp.int32),
        pltpu.VMEM((_BPW,), jnp.int32),
        [pltpu.VMEM((_UROWS, _EMB), jnp.float32)] * 2,
        [pltpu.SemaphoreType.DMA] * 2,
    ],
)
def _sc_gather(uidx_hbm, iidx_hbm, gu_hbm, gi_hbm, mu_hbm, mi_hbm,
               gu_out, gi_out, mu_out, mi_out,
               uidx_v, iidx_v, stages, sems):
    wid = lax.axis_index("s") * _NC + lax.axis_index("c")
    base = wid * _BPW

    pltpu.sync_copy(uidx_hbm.at[wid], uidx_v)
    pltpu.sync_copy(iidx_hbm.at[wid], iidx_v)

    units = []
    for table, idx_v, out in (
        (gu_hbm, uidx_v, gu_out),
        (gi_hbm, iidx_v, gi_out),
        (mu_hbm, uidx_v, mu_out),
        (mi_hbm, iidx_v, mi_out),
    ):
        for q in range(_BPW // _UROWS):
            units.append((table, idx_v, out, q))

    def fire(i):
        table, idx_v, _, q = units[i]
        stage, sem = stages[i % 2], sems[i % 2]

        def body(g, carry):
            vec = idx_v[pl.ds(q * _UROWS + g * 16, 16)]
            for j in range(16):
                pltpu.async_copy(table.at[vec[j]], stage.at[g * 16 + j], sem)
            return carry

        lax.fori_loop(0, _UROWS // 16, body, 0)

    def drain_and_flush(i):
        table, _, out, q = units[i]
        stage, sem = stages[i % 2], sems[i % 2]

        def body(d, carry):
            pltpu.make_async_copy(table.at[0], stage.at[0], sem).wait()
            return carry

        lax.fori_loop(0, _UROWS, body, 0)
        pltpu.sync_copy(stage, out.at[pl.ds(base + q * _UROWS, _UROWS)])

    fire(0)
    for i in range(len(units)):
        if i + 1 < len(units):
            fire(i + 1)
        drain_and_flush(i)


def _dense_body(gu, gi, xum, xim, gen, gW, gb, W1, b1, W2, b2, Wf, bf, out):
    xg = jnp.dot(gen[...], gW[...], preferred_element_type=jnp.float32) + gb[...]
    h = jnp.concatenate([xum[...], xim[...], xg], axis=1)
    h = jnp.maximum(
        jnp.dot(h, W1[...], preferred_element_type=jnp.float32) + b1[...], 0.0)
    h = jnp.maximum(
        jnp.dot(h, W2[...], preferred_element_type=jnp.float32) + b2[...], 0.0)
    wf = Wf[...]
    x_gmf = gu[...] * gi[...]
    acc = jnp.dot(x_gmf, wf[0:_EMB, :], preferred_element_type=jnp.float32)
    acc = acc + jnp.dot(h, wf[_EMB:, :], preferred_element_type=jnp.float32)
    out[...] = acc + bf[...]


_BT = 2048


def _dense(gu, gi, xum, xim, gen, gW, gb, W1, b1, W2, b2, Wf, bf):
    grid = (_B // _BT,)
    row = lambda i: (i, 0)
    full = lambda i: (0, 0)
    return pl.pallas_call(
        _dense_body,
        grid=grid,
        in_specs=[
            pl.BlockSpec((_BT, _EMB), row),
            pl.BlockSpec((_BT, _EMB), row),
            pl.BlockSpec((_BT, _EMB), row),
            pl.BlockSpec((_BT, _EMB), row),
            pl.BlockSpec((_BT, 18), row),
            pl.BlockSpec((18, 16), full),
            pl.BlockSpec((1, 16), full),
            pl.BlockSpec((48, 128), full),
            pl.BlockSpec((1, 128), full),
            pl.BlockSpec((128, 64), full),
            pl.BlockSpec((1, 64), full),
            pl.BlockSpec((80, 1), full),
            pl.BlockSpec((1, 1), full),
        ],
        out_specs=pl.BlockSpec((_BT, 1), row),
        out_shape=jax.ShapeDtypeStruct((_B, 1), jnp.float32),
        compiler_params=pltpu.CompilerParams(
            dimension_semantics=("parallel",)),
    )(gu, gi, xum, xim, gen, gW, gb, W1, b1, W2, b2, Wf, bf)


def kernel(user_indices, item_indices, genres_vec, gmf_user_emb, gmf_item_emb,
           mlp_user_emb, mlp_item_emb, genres_W, genres_b, W1, b1, W2, b2,
           Wf, bf):
    u2 = user_indices.astype(jnp.int32).reshape(_NW, _BPW)
    i2 = item_indices.astype(jnp.int32).reshape(_NW, _BPW)
    gu, gi, mu, mi = _sc_gather(
        u2, i2, gmf_user_emb, gmf_item_emb, mlp_user_emb, mlp_item_emb)
    out = _dense(
        gu, gi, mu, mi, genres_vec, genres_W,
        genres_b.reshape(1, -1), W1, b1.reshape(1, -1), W2,
        b2.reshape(1, -1), Wf, bf.reshape(1, -1))
    return out[:, 0]


# speedup vs baseline: 5.4100x; 3.8270x over previous
import functools

import jax
import jax.numpy as jnp
from jax import lax
from jax.experimental import pallas as pl
from jax.experimental.pallas import tpu as pltpu
from jax.experimental.pallas import tpu_sc as plsc

_B = 16384
_EMB = 16
_NC = 2
_NS = 16
_NW = _NC * _NS
_BPW = _B // _NW
_NGRP = _BPW // 16
_FBUF = 256

_mesh = plsc.VectorSubcoreMesh(core_axis_name="c", subcore_axis_name="s")


@functools.partial(
    pl.kernel,
    mesh=_mesh,
    out_type=[
        jax.ShapeDtypeStruct((_B, _EMB), jnp.float32),
        jax.ShapeDtypeStruct((_B, _EMB), jnp.float32),
        jax.ShapeDtypeStruct((_B, _EMB), jnp.float32),
        jax.ShapeDtypeStruct((_B, _EMB), jnp.float32),
    ],
    scratch_types=[
        pltpu.VMEM((_BPW,), jnp.int32),
        pltpu.VMEM((_BPW,), jnp.int32),
        [pltpu.VMEM((_EMB, 128), jnp.float32)] * 32,
        pltpu.VMEM((_FBUF, _EMB), jnp.float32),
        [pltpu.SemaphoreType.DMA] * 2,
    ],
    compiler_params=pltpu.CompilerParams(needs_layout_passes=False),
)
def _sc_gather(uidx_hbm, iidx_hbm, gu_hbm, gi_hbm, mu_hbm, mi_hbm,
               gu_out, gi_out, mu_out, mi_out,
               uidx_v, iidx_v, stages, rowbuf, sems):
    wid = lax.axis_index("s") * _NC + lax.axis_index("c")
    base = wid * _BPW

    pltpu.sync_copy(uidx_hbm.at[wid], uidx_v)
    pltpu.sync_copy(iidx_hbm.at[wid], iidx_v)

    lanes = lax.iota(jnp.int32, 16)

    for table, idx_v, out in (
        (gu_hbm, uidx_v, gu_out),
        (gi_hbm, iidx_v, gi_out),
        (mu_hbm, uidx_v, mu_out),
        (mi_hbm, iidx_v, mi_out),
    ):
        def fire(g, ring, table=table, idx_v=idx_v):
            vec = idx_v[pl.ds(g * 16, 16)]
            for j in range(16):
                u = vec[j]
                bs = pl.multiple_of((u >> 7) * 128, 128)
                pltpu.async_copy(
                    table.at[:, pl.ds(bs, 128)], stages[ring * 16 + j],
                    sems[ring])

        def extract(g, ring, table=table, idx_v=idx_v):
            for j in range(16):
                pltpu.make_async_copy(
                    table.at[:, pl.ds(0, 128)], stages[ring * 16 + j],
                    sems[ring]).wait()
            vec = idx_v[pl.ds(g * 16, 16)]
            for j in range(16):
                c = vec[j] & 127
                val = plsc.load_gather(
                    stages[ring * 16 + j], [lanes, jnp.zeros((16,), jnp.int32) + c])
                rowbuf[(g & 15) * 16 + j, :] = val

        fire(0, 0)
        fire(1, 1)

        def body(h, carry, out=out, fire=fire, extract=extract):
            g0 = 2 * h
            g1 = 2 * h + 1
            extract(g0, 0)

            @pl.when(g0 + 2 < _NGRP)
            def _():
                fire(g0 + 2, 0)

            extract(g1, 1)

            @pl.when(g1 + 2 < _NGRP)
            def _():
                fire(g1 + 2, 1)

            @pl.when((g1 & 15) == 15)
            def _():
                start = pl.multiple_of(base + ((g1 >> 4) << 8), _FBUF)
                pltpu.sync_copy(rowbuf, out.at[pl.ds(start, _FBUF)])

            return carry

        lax.fori_loop(0, _NGRP // 2, body, 0)


def _dense_body(gu, gi, xum, xim, gen, gW, gb, W1, b1, W2, b2, Wf, bf, out):
    xg = jnp.dot(gen[...], gW[...], preferred_element_type=jnp.float32) + gb[...]
    h = jnp.concatenate([xum[...], xim[...], xg], axis=1)
    h = jnp.maximum(
        jnp.dot(h, W1[...], preferred_element_type=jnp.float32) + b1[...], 0.0)
    h = jnp.maximum(
        jnp.dot(h, W2[...], preferred_element_type=jnp.float32) + b2[...], 0.0)
    wf = Wf[...]
    x_gmf = gu[...] * gi[...]
    acc = jnp.dot(x_gmf, wf[0:_EMB, :], preferred_element_type=jnp.float32)
    acc = acc + jnp.dot(h, wf[_EMB:, :], preferred_element_type=jnp.float32)
    out[...] = acc + bf[...]


_BT = 2048


def _dense(gu, gi, xum, xim, gen, gW, gb, W1, b1, W2, b2, Wf, bf):
    grid = (_B // _BT,)
    row = lambda i: (i, 0)
    full = lambda i: (0, 0)
    return pl.pallas_call(
        _dense_body,
        grid=grid,
        in_specs=[
            pl.BlockSpec((_BT, _EMB), row),
            pl.BlockSpec((_BT, _EMB), row),
            pl.BlockSpec((_BT, _EMB), row),
            pl.BlockSpec((_BT, _EMB), row),
            pl.BlockSpec((_BT, 18), row),
            pl.BlockSpec((18, 16), full),
            pl.BlockSpec((1, 16), full),
            pl.BlockSpec((48, 128), full),
            pl.BlockSpec((1, 128), full),
            pl.BlockSpec((128, 64), full),
            pl.BlockSpec((1, 64), full),
            pl.BlockSpec((80, 1), full),
            pl.BlockSpec((1, 1), full),
        ],
        out_specs=pl.BlockSpec((_BT, 1), row),
        out_shape=jax.ShapeDtypeStruct((_B, 1), jnp.float32),
        compiler_params=pltpu.CompilerParams(
            dimension_semantics=("parallel",)),
    )(gu, gi, xum, xim, gen, gW, gb, W1, b1, W2, b2, Wf, bf)


def kernel(user_indices, item_indices, genres_vec, gmf_user_emb, gmf_item_emb,
           mlp_user_emb, mlp_item_emb, genres_W, genres_b, W1, b1, W2, b2,
           Wf, bf):
    u2 = user_indices.astype(jnp.int32).reshape(_NW, _BPW)
    i2 = item_indices.astype(jnp.int32).reshape(_NW, _BPW)
    gu, gi, mu, mi = _sc_gather(
        u2, i2, gmf_user_emb.T, gmf_item_emb.T, mlp_user_emb.T,
        mlp_item_emb.T)
    out = _dense(
        gu, gi, mu, mi, genres_vec, genres_W,
        genres_b.reshape(1, -1), W1, b1.reshape(1, -1), W2,
        b2.reshape(1, -1), Wf, bf.reshape(1, -1))
    return out[:, 0]
